# baseline (speedup 1.0000x reference)
import functools

import jax
import jax.numpy as jnp
from jax import lax
from jax.experimental import pallas as pl
from jax.experimental.pallas import tpu as pltpu
from jax.experimental.pallas import tpu_sc as plsc

_N = 32768
_D = 2048
_K = 1638
_RANK = _D - _K
_NC = 2
_NS = 16
_NW = _NC * _NS
_GROUP = 16
_UNROLL = 8


def _sc_body(x_hbm, o_hbm, xin, keys, hist, ssq, in_sem, out_sem):
    lane = lax.iota(jnp.int32, 16)
    ones = jnp.ones((16,), jnp.int32)
    zeros_i = jnp.zeros((16,), jnp.int32)
    rank0 = jnp.full((16,), _RANK, jnp.int32)

    wid = lax.axis_index("s") * _NC + lax.axis_index("c")
    rows_per_worker = _N // _NW
    groups = rows_per_worker // _GROUP
    base = wid * rows_per_worker

    def in_copies(g):
        par16 = (g & 1) * _GROUP
        row0 = base + g * _GROUP
        return [(x_hbm.at[row0 + r],
                 xin.at[pl.ds(pl.multiple_of((par16 + r) * _D, _D), _D)])
                for r in range(_GROUP)]

    def out_copies(g):
        par16 = (g & 1) * _GROUP
        row0 = base + g * _GROUP
        return [(xin.at[pl.ds(pl.multiple_of((par16 + r) * _D, _D), _D)],
                 o_hbm.at[row0 + r])
                for r in range(_GROUP)]

    def _z(j, c):
        hist[j, :] = zeros_i
        return c
    lax.fori_loop(0, 256, _z, 0)

    def scan_level(rvec):
        @plsc.parallel_loop(0, 256, 1, unroll=8,
                            carry=(zeros_i, zeros_i, zeros_i))
        def res(j, c):
            cum, nb, cumlt = c
            h = hist[j, :]
            hist[j, :] = zeros_i
            cum2 = cum + h
            le = cum2 <= rvec
            return cum2, nb + jnp.where(le, 1, 0), jnp.where(le, cum2, cumlt)
        _, nb, cumlt = res
        return nb, rvec - cumlt

    for src, dst in in_copies(0):
        pltpu.async_copy(src, dst, in_sem)

    def group_body(g, c):
        par16 = (g & 1) * _GROUP
        rowflat = (lane + par16) * _D

        for src, dst in in_copies(g):
            pltpu.make_async_copy(src, dst, in_sem).wait()

        ssq[...] = jnp.zeros((16,), jnp.float32)

        @plsc.parallel_loop(0, _D, 1, unroll=_UNROLL)
        def _p1(j):
            idx = rowflat + j
            xv = plsc.load_gather(xin, [idx])
            plsc.addupdate(ssq.at[pl.ds(0, 16)], xv * xv)
            b = lax.bitcast_convert_type(xv, jnp.int32)
            s = jnp.right_shift(b, 31)
            kb = jnp.bitwise_xor(b, jnp.bitwise_and(s, jnp.int32(0x7FFFFFFF)))
            keys[pl.ds(pl.multiple_of(j * 16, 16), 16)] = kb
            d1 = jnp.right_shift(kb, 24) + 128
            plsc.addupdate_scatter(hist, [d1, lane], ones)

        b1, r1 = scan_level(rank0)
        b1s = b1 - 128

        @pl.when(g > 0)
        def _drain_out():
            for src, dst in out_copies(g - 1):
                pltpu.make_async_copy(src, dst, out_sem).wait()

        @pl.when(g + 1 < groups)
        def _prefetch():
            for src, dst in in_copies(g + 1):
                pltpu.async_copy(src, dst, in_sem)

        @plsc.parallel_loop(0, _D, 1, unroll=_UNROLL)
        def _p2(j):
            kb = keys[pl.ds(pl.multiple_of(j * 16, 16), 16)]
            m = jnp.right_shift(kb, 24) == b1s
            d2 = jnp.bitwise_and(jnp.right_shift(kb, 16), 255)
            plsc.addupdate_scatter(hist, [d2, lane], ones, mask=m)

        b2, r2 = scan_level(r1)
        p2s = b1s * 256 + b2

        @plsc.parallel_loop(0, _D, 1, unroll=_UNROLL)
        def _p3(j):
            kb = keys[pl.ds(pl.multiple_of(j * 16, 16), 16)]
            m = jnp.right_shift(kb, 16) == p2s
            d3 = jnp.bitwise_and(jnp.right_shift(kb, 8), 255)
            plsc.addupdate_scatter(hist, [d3, lane], ones, mask=m)

        b3, _ = scan_level(r2)
        p3s = p2s * 256 + b3
        ks_t = p3s * 256

        tf = lax.bitcast_convert_type(
            jnp.bitwise_xor(
                ks_t,
                jnp.bitwise_and(jnp.right_shift(ks_t, 31),
                                jnp.int32(0x7FFFFFFF))),
            jnp.float32)

        s = ssq[...]
        y = lax.bitcast_convert_type(
            jnp.int32(0x5F3759DF)
            - jnp.right_shift(lax.bitcast_convert_type(s, jnp.int32), 1),
            jnp.float32)
        half = 0.5 * s
        for _ in range(3):
            y = y * (1.5 - half * y * y)
        boost = jnp.exp(jnp.float32(_K) * y)

        @plsc.parallel_loop(0, _D, 1, unroll=_UNROLL)
        def _po(j):
            idx = rowflat + j
            xv = plsc.load_gather(xin, [idx])
            ov = jnp.where(xv >= tf, xv * boost, jnp.float32(0.0))
            plsc.store_scatter(xin, [idx], ov)

        for src, dst in out_copies(g):
            pltpu.async_copy(src, dst, out_sem)
        return c

    lax.fori_loop(0, groups, group_body, 0)

    for src, dst in out_copies(groups - 1):
        pltpu.make_async_copy(src, dst, out_sem).wait()


@functools.partial(jax.jit, static_argnames=())
def kernel(inputs):
    n, d = inputs.shape
    assert (n, d) == (_N, _D)
    mesh = plsc.VectorSubcoreMesh(
        core_axis_name="c", subcore_axis_name="s",
        num_cores=_NC, num_subcores=_NS)
    f = pl.kernel(
        _sc_body,
        out_type=jax.ShapeDtypeStruct((_N, _D), jnp.float32),
        mesh=mesh,
        scratch_types=[
            pltpu.VMEM((2 * _GROUP * _D,), jnp.float32),
            pltpu.VMEM((_D * 16,), jnp.int32),
            pltpu.VMEM((256, 16), jnp.int32),
            pltpu.VMEM((16,), jnp.float32),
            pltpu.SemaphoreType.DMA,
            pltpu.SemaphoreType.DMA,
        ],
        compiler_params=pltpu.CompilerParams(
            use_tc_tiling_on_sc=False, needs_layout_passes=False),
    )
    return f(inputs)


# speedup vs baseline: 2.3034x; 2.3034x over previous
import functools

import jax
import jax.numpy as jnp
from jax import lax
from jax.experimental import pallas as pl
from jax.experimental.pallas import tpu as pltpu
from jax.experimental.pallas import tpu_sc as plsc

_N = 32768
_D = 2048
_K = 1638
_RANK = _D - _K
_NC = 2
_NS = 16
_NW = _NC * _NS
_GROUP = 16
_UNROLL = 8


def _sc_body(x_hbm, o_hbm, xin, keys, hist, ssq, in_sem, out_sem):
    lane = lax.iota(jnp.int32, 16)
    ones = jnp.ones((16,), jnp.int32)
    zeros_i = jnp.zeros((16,), jnp.int32)
    rank0 = jnp.full((16,), _RANK, jnp.int32)

    wid = lax.axis_index("s") * _NC + lax.axis_index("c")
    rows_per_worker = _N // _NW
    groups = rows_per_worker // _GROUP
    base = wid * rows_per_worker

    def in_copies(g):
        par16 = (g & 1) * _GROUP
        row0 = base + g * _GROUP
        return [(x_hbm.at[row0 + r],
                 xin.at[pl.ds(pl.multiple_of((par16 + r) * _D, _D), _D)])
                for r in range(_GROUP)]

    def out_copies(g):
        par16 = (g & 1) * _GROUP
        row0 = base + g * _GROUP
        return [(xin.at[pl.ds(pl.multiple_of((par16 + r) * _D, _D), _D)],
                 o_hbm.at[row0 + r])
                for r in range(_GROUP)]

    def _z(j, c):
        hist[j, :] = zeros_i
        return c
    lax.fori_loop(0, 256, _z, 0)

    def scan_level(rvec):
        @plsc.parallel_loop(0, 256, 1, unroll=8,
                            carry=(zeros_i, zeros_i, zeros_i))
        def res(j, c):
            cum, nb, cumlt = c
            h = hist[j, :]
            hist[j, :] = zeros_i
            cum2 = cum + h
            le = cum2 <= rvec
            return cum2, nb + jnp.where(le, 1, 0), jnp.where(le, cum2, cumlt)
        _, nb, cumlt = res
        return nb, rvec - cumlt

    for src, dst in in_copies(0):
        pltpu.async_copy(src, dst, in_sem)

    def group_body(g, c):
        par16 = (g & 1) * _GROUP
        rowflat = (lane + par16) * _D

        for src, dst in in_copies(g):
            pltpu.make_async_copy(src, dst, in_sem).wait()

        ssq[...] = jnp.zeros((16,), jnp.float32)

        @plsc.parallel_loop(0, _D, 1, unroll=_UNROLL)
        def _p1(j):
            idx = rowflat + jnp.bitwise_and(lane + j, _D - 1)
            xv = plsc.load_gather(xin, [idx])
            plsc.addupdate(ssq.at[pl.ds(0, 16)], xv * xv)
            b = lax.bitcast_convert_type(xv, jnp.int32)
            s = jnp.right_shift(b, 31)
            kb = jnp.bitwise_xor(b, jnp.bitwise_and(s, jnp.int32(0x7FFFFFFF)))
            keys[pl.ds(pl.multiple_of(j * 16, 16), 16)] = kb
            d1 = jnp.right_shift(kb, 24) + 128
            plsc.addupdate_scatter(hist, [d1, lane], ones)

        b1, r1 = scan_level(rank0)
        b1s = b1 - 128

        @pl.when(g > 0)
        def _drain_out():
            for src, dst in out_copies(g - 1):
                pltpu.make_async_copy(src, dst, out_sem).wait()

        @pl.when(g + 1 < groups)
        def _prefetch():
            for src, dst in in_copies(g + 1):
                pltpu.async_copy(src, dst, in_sem)

        @plsc.parallel_loop(0, _D, 1, unroll=_UNROLL)
        def _p2(j):
            kb = keys[pl.ds(pl.multiple_of(j * 16, 16), 16)]
            m = jnp.right_shift(kb, 24) == b1s
            d2 = jnp.bitwise_and(jnp.right_shift(kb, 16), 255)
            plsc.addupdate_scatter(hist, [d2, lane], ones, mask=m)

        b2, r2 = scan_level(r1)
        p2s = b1s * 256 + b2

        @plsc.parallel_loop(0, _D, 1, unroll=_UNROLL)
        def _p3(j):
            kb = keys[pl.ds(pl.multiple_of(j * 16, 16), 16)]
            m = jnp.right_shift(kb, 16) == p2s
            d3 = jnp.bitwise_and(jnp.right_shift(kb, 8), 255)
            plsc.addupdate_scatter(hist, [d3, lane], ones, mask=m)

        b3, _ = scan_level(r2)
        p3s = p2s * 256 + b3
        ks_t = p3s * 256

        tf = lax.bitcast_convert_type(
            jnp.bitwise_xor(
                ks_t,
                jnp.bitwise_and(jnp.right_shift(ks_t, 31),
                                jnp.int32(0x7FFFFFFF))),
            jnp.float32)

        s = ssq[...]
        y = lax.bitcast_convert_type(
            jnp.int32(0x5F3759DF)
            - jnp.right_shift(lax.bitcast_convert_type(s, jnp.int32), 1),
            jnp.float32)
        half = 0.5 * s
        for _ in range(3):
            y = y * (1.5 - half * y * y)
        boost = jnp.exp(jnp.float32(_K) * y)

        @plsc.parallel_loop(0, _D, 1, unroll=_UNROLL)
        def _po(j):
            idx = rowflat + jnp.bitwise_and(lane + j, _D - 1)
            xv = plsc.load_gather(xin, [idx])
            ov = jnp.where(xv >= tf, xv * boost, jnp.float32(0.0))
            plsc.store_scatter(xin, [idx], ov)

        for src, dst in out_copies(g):
            pltpu.async_copy(src, dst, out_sem)
        return c

    lax.fori_loop(0, groups, group_body, 0)

    for src, dst in out_copies(groups - 1):
        pltpu.make_async_copy(src, dst, out_sem).wait()


@functools.partial(jax.jit, static_argnames=())
def kernel(inputs):
    n, d = inputs.shape
    assert (n, d) == (_N, _D)
    mesh = plsc.VectorSubcoreMesh(
        core_axis_name="c", subcore_axis_name="s",
        num_cores=_NC, num_subcores=_NS)
    f = pl.kernel(
        _sc_body,
        out_type=jax.ShapeDtypeStruct((_N, _D), jnp.float32),
        mesh=mesh,
        scratch_types=[
            pltpu.VMEM((2 * _GROUP * _D,), jnp.float32),
            pltpu.VMEM((_D * 16,), jnp.int32),
            pltpu.VMEM((256, 16), jnp.int32),
            pltpu.VMEM((16,), jnp.float32),
            pltpu.SemaphoreType.DMA,
            pltpu.SemaphoreType.DMA,
        ],
        compiler_params=pltpu.CompilerParams(
            use_tc_tiling_on_sc=False, needs_layout_passes=False),
    )
    return f(inputs)


# speedup vs baseline: 2.3092x; 1.0025x over previous
import functools

import jax
import jax.numpy as jnp
from jax import lax
from jax.experimental import pallas as pl
from jax.experimental.pallas import tpu as pltpu
from jax.experimental.pallas import tpu_sc as plsc

_N = 32768
_D = 2048
_K = 1638
_RANK = _D - _K
_NC = 2
_NS = 16
_NW = _NC * _NS
_GROUP = 16
_UNROLL = 16


def _sc_body(x_hbm, o_hbm, xin, keys, hist, ssq, in_sem, out_sem):
    lane = lax.iota(jnp.int32, 16)
    ones = jnp.ones((16,), jnp.int32)
    zeros_i = jnp.zeros((16,), jnp.int32)
    rank0 = jnp.full((16,), _RANK, jnp.int32)

    wid = lax.axis_index("s") * _NC + lax.axis_index("c")
    rows_per_worker = _N // _NW
    groups = rows_per_worker // _GROUP
    base = wid * rows_per_worker

    def in_copies(g):
        par16 = (g & 1) * _GROUP
        row0 = base + g * _GROUP
        return [(x_hbm.at[row0 + r],
                 xin.at[pl.ds(pl.multiple_of((par16 + r) * _D, _D), _D)])
                for r in range(_GROUP)]

    def out_copies(g):
        par16 = (g & 1) * _GROUP
        row0 = base + g * _GROUP
        return [(xin.at[pl.ds(pl.multiple_of((par16 + r) * _D, _D), _D)],
                 o_hbm.at[row0 + r])
                for r in range(_GROUP)]

    def _z(j, c):
        hist[j, :] = zeros_i
        return c
    lax.fori_loop(0, 256, _z, 0)

    def scan_level(rvec):
        @plsc.parallel_loop(0, 256, 1, unroll=8,
                            carry=(zeros_i, zeros_i, zeros_i))
        def res(j, c):
            cum, nb, cumlt = c
            h = hist[j, :]
            hist[j, :] = zeros_i
            cum2 = cum + h
            le = cum2 <= rvec
            return cum2, nb + jnp.where(le, 1, 0), jnp.where(le, cum2, cumlt)
        _, nb, cumlt = res
        return nb, rvec - cumlt

    for src, dst in in_copies(0):
        pltpu.async_copy(src, dst, in_sem)

    def group_body(g, c):
        par16 = (g & 1) * _GROUP
        rowflat = (lane + par16) * _D

        for src, dst in in_copies(g):
            pltpu.make_async_copy(src, dst, in_sem).wait()

        ssq[...] = jnp.zeros((16,), jnp.float32)

        @plsc.parallel_loop(0, _D, 1, unroll=_UNROLL)
        def _p1(j):
            idx = rowflat + jnp.bitwise_and(lane + j, _D - 1)
            xv = plsc.load_gather(xin, [idx])
            plsc.addupdate(ssq.at[pl.ds(0, 16)], xv * xv)
            b = lax.bitcast_convert_type(xv, jnp.int32)
            s = jnp.right_shift(b, 31)
            kb = jnp.bitwise_xor(b, jnp.bitwise_and(s, jnp.int32(0x7FFFFFFF)))
            keys[pl.ds(pl.multiple_of(j * 16, 16), 16)] = kb
            d1 = jnp.right_shift(kb, 24) + 128
            plsc.addupdate_scatter(hist, [d1, lane], ones)

        b1, r1 = scan_level(rank0)
        b1s = b1 - 128

        @pl.when(g > 0)
        def _drain_out():
            for src, dst in out_copies(g - 1):
                pltpu.make_async_copy(src, dst, out_sem).wait()

        @pl.when(g + 1 < groups)
        def _prefetch():
            for src, dst in in_copies(g + 1):
                pltpu.async_copy(src, dst, in_sem)

        @plsc.parallel_loop(0, _D, 1, unroll=_UNROLL)
        def _p2(j):
            kb = keys[pl.ds(pl.multiple_of(j * 16, 16), 16)]
            m = jnp.right_shift(kb, 24) == b1s
            d2 = jnp.bitwise_and(jnp.right_shift(kb, 16), 255)
            plsc.addupdate_scatter(hist, [d2, lane], ones, mask=m)

        b2, r2 = scan_level(r1)
        p2s = b1s * 256 + b2

        @plsc.parallel_loop(0, _D, 1, unroll=_UNROLL)
        def _p3(j):
            kb = keys[pl.ds(pl.multiple_of(j * 16, 16), 16)]
            m = jnp.right_shift(kb, 16) == p2s
            d3 = jnp.bitwise_and(jnp.right_shift(kb, 8), 255)
            plsc.addupdate_scatter(hist, [d3, lane], ones, mask=m)

        b3, _ = scan_level(r2)
        p3s = p2s * 256 + b3
        ks_t = p3s * 256

        tf = lax.bitcast_convert_type(
            jnp.bitwise_xor(
                ks_t,
                jnp.bitwise_and(jnp.right_shift(ks_t, 31),
                                jnp.int32(0x7FFFFFFF))),
            jnp.float32)

        s = ssq[...]
        y = lax.bitcast_convert_type(
            jnp.int32(0x5F3759DF)
            - jnp.right_shift(lax.bitcast_convert_type(s, jnp.int32), 1),
            jnp.float32)
        half = 0.5 * s
        for _ in range(3):
            y = y * (1.5 - half * y * y)
        boost = jnp.exp(jnp.float32(_K) * y)

        @plsc.parallel_loop(0, _D, 1, unroll=_UNROLL)
        def _po(j):
            idx = rowflat + jnp.bitwise_and(lane + j, _D - 1)
            xv = plsc.load_gather(xin, [idx])
            ov = jnp.where(xv >= tf, xv * boost, jnp.float32(0.0))
            plsc.store_scatter(xin, [idx], ov)

        for src, dst in out_copies(g):
            pltpu.async_copy(src, dst, out_sem)
        return c

    lax.fori_loop(0, groups, group_body, 0)

    for src, dst in out_copies(groups - 1):
        pltpu.make_async_copy(src, dst, out_sem).wait()


@functools.partial(jax.jit, static_argnames=())
def kernel(inputs):
    n, d = inputs.shape
    assert (n, d) == (_N, _D)
    mesh = plsc.VectorSubcoreMesh(
        core_axis_name="c", subcore_axis_name="s",
        num_cores=_NC, num_subcores=_NS)
    f = pl.kernel(
        _sc_body,
        out_type=jax.ShapeDtypeStruct((_N, _D), jnp.float32),
        mesh=mesh,
        scratch_types=[
            pltpu.VMEM((2 * _GROUP * _D,), jnp.float32),
            pltpu.VMEM((_D * 16,), jnp.int32),
            pltpu.VMEM((256, 16), jnp.int32),
            pltpu.VMEM((16,), jnp.float32),
            pltpu.SemaphoreType.DMA,
            pltpu.SemaphoreType.DMA,
        ],
        compiler_params=pltpu.CompilerParams(
            use_tc_tiling_on_sc=False, needs_layout_passes=False,
            skip_device_barrier=True),
    )
    return f(inputs)


# speedup vs baseline: 3.4335x; 1.4869x over previous
import functools

import jax
import jax.numpy as jnp
from jax import lax
from jax.experimental import pallas as pl
from jax.experimental.pallas import tpu as pltpu
from jax.experimental.pallas import tpu_sc as plsc

_N = 32768
_D = 2048
_K = 1638
_RANK = _D - _K
_NC = 2
_NS = 16
_NW = _NC * _NS
_GROUP = 16
_DC = 1792
_UNROLL = 16


def _sc_body(x_hbm, o_hbm, xin, keys, hist, ssq, in_sem, out_sem):
    lane = lax.iota(jnp.int32, 16)
    ones = jnp.ones((16,), jnp.int32)
    zeros_i = jnp.zeros((16,), jnp.int32)
    rank0 = jnp.full((16,), _RANK, jnp.int32)

    wid = lax.axis_index("s") * _NC + lax.axis_index("c")
    rows_per_worker = _N // _NW
    groups = rows_per_worker // _GROUP
    base = wid * rows_per_worker

    def in_copy(g, r):
        par16 = (g & 1) * _GROUP
        row0 = base + g * _GROUP
        return (x_hbm.at[row0 + r],
                xin.at[pl.ds(pl.multiple_of((par16 + r) * _D, _D), _D)])

    def out_copy(g, r):
        par16 = (g & 1) * _GROUP
        row0 = base + g * _GROUP
        return (xin.at[pl.ds(pl.multiple_of((par16 + r) * _D, _D), _D)],
                o_hbm.at[row0 + r])

    def issue_in(g):
        def body(r, c):
            src, dst = in_copy(g, r)
            pltpu.async_copy(src, dst, in_sem)
            return c
        lax.fori_loop(0, _GROUP, body, 0)

    def wait_in(g):
        def body(r, c):
            src, dst = in_copy(g, r)
            pltpu.make_async_copy(src, dst, in_sem).wait()
            return c
        lax.fori_loop(0, _GROUP, body, 0)

    def issue_out(g):
        def body(r, c):
            src, dst = out_copy(g, r)
            pltpu.async_copy(src, dst, out_sem)
            return c
        lax.fori_loop(0, _GROUP, body, 0)

    def wait_out(g):
        def body(r, c):
            src, dst = out_copy(g, r)
            pltpu.make_async_copy(src, dst, out_sem).wait()
            return c
        lax.fori_loop(0, _GROUP, body, 0)

    def _z(j, c):
        hist[j, :] = zeros_i
        return c
    lax.fori_loop(0, 256, _z, 0)

    def scan_level(rvec):
        @plsc.parallel_loop(0, 256, 1, unroll=8,
                            carry=(zeros_i, zeros_i, zeros_i))
        def res(j, c):
            cum, nb, cumlt = c
            h = hist[j, :]
            hist[j, :] = zeros_i
            cum2 = cum + h
            le = cum2 <= rvec
            return cum2, nb + jnp.where(le, 1, 0), jnp.where(le, cum2, cumlt)
        _, nb, cumlt = res
        return nb, rvec - cumlt

    issue_in(0)

    def group_body(g, c):
        par16 = (g & 1) * _GROUP
        rowflat = (lane + par16) * _D

        wait_in(g)

        ssq[...] = jnp.zeros((16,), jnp.float32)

        @plsc.parallel_loop(0, _DC, 1, unroll=_UNROLL)
        def _p1(j):
            idx = rowflat + jnp.bitwise_and(lane + j, _D - 1)
            xv = plsc.load_gather(xin, [idx])
            plsc.addupdate(ssq.at[pl.ds(0, 16)], xv * xv)
            b = lax.bitcast_convert_type(xv, jnp.int32)
            s = jnp.right_shift(b, 31)
            kb = jnp.bitwise_xor(b, jnp.bitwise_and(s, jnp.int32(0x7FFFFFFF)))
            keys[pl.ds(pl.multiple_of(j * 16, 16), 16)] = kb
            d1 = jnp.right_shift(kb, 24) + 128
            plsc.addupdate_scatter(hist, [d1, lane], ones)

        @plsc.parallel_loop(_DC, _D, 1, unroll=_UNROLL)
        def _p1t(j):
            idx = rowflat + jnp.bitwise_and(lane + j, _D - 1)
            xv = plsc.load_gather(xin, [idx])
            plsc.addupdate(ssq.at[pl.ds(0, 16)], xv * xv)
            b = lax.bitcast_convert_type(xv, jnp.int32)
            s = jnp.right_shift(b, 31)
            kb = jnp.bitwise_xor(b, jnp.bitwise_and(s, jnp.int32(0x7FFFFFFF)))
            d1 = jnp.right_shift(kb, 24) + 128
            plsc.addupdate_scatter(hist, [d1, lane], ones)

        b1, r1 = scan_level(rank0)
        b1s = b1 - 128

        @pl.when(g > 0)
        def _drain_out():
            wait_out(g - 1)

        @pl.when(g + 1 < groups)
        def _prefetch():
            issue_in(g + 1)

        @plsc.parallel_loop(0, _DC, 1, unroll=_UNROLL)
        def _p2(j):
            kb = keys[pl.ds(pl.multiple_of(j * 16, 16), 16)]
            m = jnp.right_shift(kb, 24) == b1s
            d2 = jnp.bitwise_and(jnp.right_shift(kb, 16), 255)
            plsc.addupdate_scatter(hist, [d2, lane], ones, mask=m)

        @plsc.parallel_loop(_DC, _D, 1, unroll=_UNROLL)
        def _p2t(j):
            idx = rowflat + jnp.bitwise_and(lane + j, _D - 1)
            xv = plsc.load_gather(xin, [idx])
            b = lax.bitcast_convert_type(xv, jnp.int32)
            s = jnp.right_shift(b, 31)
            kb = jnp.bitwise_xor(b, jnp.bitwise_and(s, jnp.int32(0x7FFFFFFF)))
            m = jnp.right_shift(kb, 24) == b1s
            d2 = jnp.bitwise_and(jnp.right_shift(kb, 16), 255)
            plsc.addupdate_scatter(hist, [d2, lane], ones, mask=m)

        b2, r2 = scan_level(r1)
        p2s = b1s * 256 + b2

        @plsc.parallel_loop(0, _DC, 1, unroll=_UNROLL)
        def _p3(j):
            kb = keys[pl.ds(pl.multiple_of(j * 16, 16), 16)]
            m = jnp.right_shift(kb, 16) == p2s
            d3 = jnp.bitwise_and(jnp.right_shift(kb, 8), 255)
            plsc.addupdate_scatter(hist, [d3, lane], ones, mask=m)

        @plsc.parallel_loop(_DC, _D, 1, unroll=_UNROLL)
        def _p3t(j):
            idx = rowflat + jnp.bitwise_and(lane + j, _D - 1)
            xv = plsc.load_gather(xin, [idx])
            b = lax.bitcast_convert_type(xv, jnp.int32)
            s = jnp.right_shift(b, 31)
            kb = jnp.bitwise_xor(b, jnp.bitwise_and(s, jnp.int32(0x7FFFFFFF)))
            m = jnp.right_shift(kb, 16) == p2s
            d3 = jnp.bitwise_and(jnp.right_shift(kb, 8), 255)
            plsc.addupdate_scatter(hist, [d3, lane], ones, mask=m)

        b3, _ = scan_level(r2)
        p3s = p2s * 256 + b3
        ks_t = p3s * 256

        tf = lax.bitcast_convert_type(
            jnp.bitwise_xor(
                ks_t,
                jnp.bitwise_and(jnp.right_shift(ks_t, 31),
                                jnp.int32(0x7FFFFFFF))),
            jnp.float32)

        s = ssq[...]
        y = lax.bitcast_convert_type(
            jnp.int32(0x5F3759DF)
            - jnp.right_shift(lax.bitcast_convert_type(s, jnp.int32), 1),
            jnp.float32)
        half = 0.5 * s
        for _ in range(3):
            y = y * (1.5 - half * y * y)
        boost = jnp.exp(jnp.float32(_K) * y)

        @plsc.parallel_loop(0, _D, 1, unroll=_UNROLL)
        def _po(j):
            idx = rowflat + jnp.bitwise_and(lane + j, _D - 1)
            xv = plsc.load_gather(xin, [idx])
            ov = jnp.where(xv >= tf, xv * boost, jnp.float32(0.0))
            plsc.store_scatter(xin, [idx], ov)

        issue_out(g)
        return c

    lax.fori_loop(0, groups, group_body, 0)

    wait_out(groups - 1)


@functools.partial(jax.jit, static_argnames=())
def kernel(inputs):
    n, d = inputs.shape
    assert (n, d) == (_N, _D)
    mesh = plsc.VectorSubcoreMesh(
        core_axis_name="c", subcore_axis_name="s",
        num_cores=_NC, num_subcores=_NS)
    f = pl.kernel(
        _sc_body,
        out_type=jax.ShapeDtypeStruct((_N, _D), jnp.float32),
        mesh=mesh,
        scratch_types=[
            pltpu.VMEM((2 * _GROUP * _D,), jnp.float32),
            pltpu.VMEM((_DC * 16,), jnp.int32),
            pltpu.VMEM((256, 16), jnp.int32),
            pltpu.VMEM((16,), jnp.float32),
            pltpu.SemaphoreType.DMA,
            pltpu.SemaphoreType.DMA,
        ],
        compiler_params=pltpu.CompilerParams(
            use_tc_tiling_on_sc=True, needs_layout_passes=False,
            skip_device_barrier=True),
    )
    return f(inputs)


# speedup vs baseline: 3.4892x; 1.0162x over previous
import functools

import jax
import jax.numpy as jnp
from jax import lax
from jax.experimental import pallas as pl
from jax.experimental.pallas import tpu as pltpu
from jax.experimental.pallas import tpu_sc as plsc

_N = 32768
_D = 2048
_K = 1638
_RANK = _D - _K
_NC = 2
_NS = 16
_NW = _NC * _NS
_GROUP = 16
_DC = 1920
_UNROLL = 16


def _sc_body(x_hbm, o_hbm, xin, keys, hist, ssq, in_sem, out_sem):
    lane = lax.iota(jnp.int32, 16)
    ones = jnp.ones((16,), jnp.int32)
    zeros_i = jnp.zeros((16,), jnp.int32)
    rank0 = jnp.full((16,), _RANK, jnp.int32)

    wid = lax.axis_index("s") * _NC + lax.axis_index("c")
    rows_per_worker = _N // _NW
    groups = rows_per_worker // _GROUP
    base = wid * rows_per_worker

    def in_copy(g, r):
        par16 = (g & 1) * _GROUP
        row0 = base + g * _GROUP
        return (x_hbm.at[row0 + r],
                xin.at[pl.ds(pl.multiple_of((par16 + r) * _D, _D), _D)])

    def out_copy(g, r):
        par16 = (g & 1) * _GROUP
        row0 = base + g * _GROUP
        return (xin.at[pl.ds(pl.multiple_of((par16 + r) * _D, _D), _D)],
                o_hbm.at[row0 + r])

    def issue_in(g):
        def body(r, c):
            src, dst = in_copy(g, r)
            pltpu.async_copy(src, dst, in_sem)
            return c
        lax.fori_loop(0, _GROUP, body, 0)

    def wait_in(g):
        def body(r, c):
            src, dst = in_copy(g, r)
            pltpu.make_async_copy(src, dst, in_sem).wait()
            return c
        lax.fori_loop(0, _GROUP, body, 0)

    def issue_out(g):
        def body(r, c):
            src, dst = out_copy(g, r)
            pltpu.async_copy(src, dst, out_sem)
            return c
        lax.fori_loop(0, _GROUP, body, 0)

    def wait_out(g):
        def body(r, c):
            src, dst = out_copy(g, r)
            pltpu.make_async_copy(src, dst, out_sem).wait()
            return c
        lax.fori_loop(0, _GROUP, body, 0)

    def _z(j, c):
        hist[j, :] = zeros_i
        return c
    lax.fori_loop(0, 256, _z, 0)

    def scan_level(rvec):
        @plsc.parallel_loop(0, 256, 1, unroll=8,
                            carry=(zeros_i, zeros_i, zeros_i))
        def res(j, c):
            cum, nb, cumlt = c
            h = hist[j, :]
            hist[j, :] = zeros_i
            cum2 = cum + h
            le = cum2 <= rvec
            return cum2, nb + jnp.where(le, 1, 0), jnp.where(le, cum2, cumlt)
        _, nb, cumlt = res
        return nb, rvec - cumlt

    issue_in(0)

    def group_body(g, c):
        par16 = (g & 1) * _GROUP
        rowflat = (lane + par16) * _D

        wait_in(g)

        ssq[...] = jnp.zeros((16,), jnp.float32)

        @plsc.parallel_loop(0, _DC, 1, unroll=_UNROLL)
        def _p1(j):
            idx = rowflat + jnp.bitwise_and(lane + j, _D - 1)
            xv = plsc.load_gather(xin, [idx])
            plsc.addupdate(ssq.at[pl.ds(0, 16)], xv * xv)
            b = lax.bitcast_convert_type(xv, jnp.int32)
            s = jnp.right_shift(b, 31)
            kb = jnp.bitwise_xor(b, jnp.bitwise_and(s, jnp.int32(0x7FFFFFFF)))
            keys[pl.ds(pl.multiple_of(j * 16, 16), 16)] = kb
            d1 = jnp.right_shift(kb, 24) + 128
            plsc.addupdate_scatter(hist, [d1, lane], ones)

        @plsc.parallel_loop(_DC, _D, 1, unroll=_UNROLL)
        def _p1t(j):
            idx = rowflat + jnp.bitwise_and(lane + j, _D - 1)
            xv = plsc.load_gather(xin, [idx])
            plsc.addupdate(ssq.at[pl.ds(0, 16)], xv * xv)
            b = lax.bitcast_convert_type(xv, jnp.int32)
            s = jnp.right_shift(b, 31)
            kb = jnp.bitwise_xor(b, jnp.bitwise_and(s, jnp.int32(0x7FFFFFFF)))
            d1 = jnp.right_shift(kb, 24) + 128
            plsc.addupdate_scatter(hist, [d1, lane], ones)

        b1, r1 = scan_level(rank0)
        b1s = b1 - 128

        @pl.when(g > 0)
        def _drain_out():
            wait_out(g - 1)

        @pl.when(g + 1 < groups)
        def _prefetch():
            issue_in(g + 1)

        @plsc.parallel_loop(0, _DC, 1, unroll=_UNROLL)
        def _p2(j):
            kb = keys[pl.ds(pl.multiple_of(j * 16, 16), 16)]
            m = jnp.right_shift(kb, 24) == b1s
            d2 = jnp.bitwise_and(jnp.right_shift(kb, 16), 255)
            plsc.addupdate_scatter(hist, [d2, lane], ones, mask=m)

        @plsc.parallel_loop(_DC, _D, 1, unroll=_UNROLL)
        def _p2t(j):
            idx = rowflat + jnp.bitwise_and(lane + j, _D - 1)
            xv = plsc.load_gather(xin, [idx])
            b = lax.bitcast_convert_type(xv, jnp.int32)
            s = jnp.right_shift(b, 31)
            kb = jnp.bitwise_xor(b, jnp.bitwise_and(s, jnp.int32(0x7FFFFFFF)))
            m = jnp.right_shift(kb, 24) == b1s
            d2 = jnp.bitwise_and(jnp.right_shift(kb, 16), 255)
            plsc.addupdate_scatter(hist, [d2, lane], ones, mask=m)

        b2, r2 = scan_level(r1)
        p2s = b1s * 256 + b2

        @plsc.parallel_loop(0, _DC, 1, unroll=_UNROLL)
        def _p3(j):
            kb = keys[pl.ds(pl.multiple_of(j * 16, 16), 16)]
            m = jnp.right_shift(kb, 16) == p2s
            d3 = jnp.bitwise_and(jnp.right_shift(kb, 8), 255)
            plsc.addupdate_scatter(hist, [d3, lane], ones, mask=m)

        @plsc.parallel_loop(_DC, _D, 1, unroll=_UNROLL)
        def _p3t(j):
            idx = rowflat + jnp.bitwise_and(lane + j, _D - 1)
            xv = plsc.load_gather(xin, [idx])
            b = lax.bitcast_convert_type(xv, jnp.int32)
            s = jnp.right_shift(b, 31)
            kb = jnp.bitwise_xor(b, jnp.bitwise_and(s, jnp.int32(0x7FFFFFFF)))
            m = jnp.right_shift(kb, 16) == p2s
            d3 = jnp.bitwise_and(jnp.right_shift(kb, 8), 255)
            plsc.addupdate_scatter(hist, [d3, lane], ones, mask=m)

        b3, _ = scan_level(r2)
        p3s = p2s * 256 + b3
        ks_t = p3s * 256

        tf = lax.bitcast_convert_type(
            jnp.bitwise_xor(
                ks_t,
                jnp.bitwise_and(jnp.right_shift(ks_t, 31),
                                jnp.int32(0x7FFFFFFF))),
            jnp.float32)

        s = ssq[...]
        y = lax.bitcast_convert_type(
            jnp.int32(0x5F3759DF)
            - jnp.right_shift(lax.bitcast_convert_type(s, jnp.int32), 1),
            jnp.float32)
        half = 0.5 * s
        for _ in range(3):
            y = y * (1.5 - half * y * y)
        boost = jnp.exp(jnp.float32(_K) * y)

        @plsc.parallel_loop(0, _D, 1, unroll=_UNROLL)
        def _po(j):
            idx = rowflat + jnp.bitwise_and(lane + j, _D - 1)
            xv = plsc.load_gather(xin, [idx])
            ov = jnp.where(xv >= tf, xv * boost, jnp.float32(0.0))
            plsc.store_scatter(xin, [idx], ov)

        issue_out(g)
        return c

    lax.fori_loop(0, groups, group_body, 0)

    wait_out(groups - 1)


@functools.partial(jax.jit, static_argnames=())
def kernel(inputs):
    n, d = inputs.shape
    assert (n, d) == (_N, _D)
    mesh = plsc.VectorSubcoreMesh(
        core_axis_name="c", subcore_axis_name="s",
        num_cores=_NC, num_subcores=_NS)
    f = pl.kernel(
        _sc_body,
        out_type=jax.ShapeDtypeStruct((_N, _D), jnp.float32),
        mesh=mesh,
        scratch_types=[
            pltpu.VMEM((2 * _GROUP * _D,), jnp.float32),
            pltpu.VMEM((_DC * 16,), jnp.int32),
            pltpu.VMEM((256, 16), jnp.int32),
            pltpu.VMEM((16,), jnp.float32),
            pltpu.SemaphoreType.DMA,
            pltpu.SemaphoreType.DMA,
        ],
        compiler_params=pltpu.CompilerParams(
            use_tc_tiling_on_sc=True, needs_layout_passes=False,
            skip_device_barrier=True),
    )
    return f(inputs)
